# Initial kernel scaffold; baseline (speedup 1.0000x reference)
import jax
import jax.numpy as jnp
from jax.experimental import pallas as pl


def kernel(user, item, mf_user_em, mf_item_em, mlp_user_em, mlp_item_em, W1, b1, W2, b2, Wout):
    raise NotImplementedError("write your pallas kernel here")


# speedup vs baseline: 1.1839x; 1.1839x over previous
import functools

import jax
import jax.numpy as jnp
from jax import lax
from jax.experimental import pallas as pl
from jax.experimental.pallas import tpu as pltpu
from jax.experimental.pallas import tpu_sc as plsc

_NC = 2
_NS = 16
_NW = _NC * _NS

_CH = 128


def _sc_gather(item_idx, user_idx, mf_item_em, mlp_item_em, mf_user_em,
               mlp_user_em, n_item_chunks):
  nw, nch, ch = item_idx.shape
  assert nch == n_item_chunks and ch == _CH
  n_users_w = user_idx.shape[1]
  d = mf_item_em.shape[1]
  bl = nw * nch * ch

  mesh = plsc.VectorSubcoreMesh(core_axis_name="c", subcore_axis_name="s",
                                num_cores=_NC, num_subcores=_NS)

  @functools.partial(
      pl.kernel,
      out_type=[
          jax.ShapeDtypeStruct((bl, d), jnp.float32),
          jax.ShapeDtypeStruct((bl, d), jnp.float32),
          jax.ShapeDtypeStruct((nw * n_users_w, d), jnp.float32),
          jax.ShapeDtypeStruct((nw * n_users_w, d), jnp.float32),
      ],
      mesh=mesh,
      compiler_params=pltpu.CompilerParams(use_tc_tiling_on_sc=False),
      scratch_types=[
          pltpu.VMEM((nch, ch), jnp.int32),
          pltpu.VMEM((n_users_w,), jnp.int32),
          pltpu.VMEM((ch, d), jnp.float32),
          pltpu.VMEM((ch, d), jnp.float32),
          pltpu.SemaphoreType.DMA,
          pltpu.SemaphoreType.DMA,
      ],
  )
  def k(item_idx_hbm, user_idx_hbm, mf_item_hbm, mlp_item_hbm, mf_user_hbm,
        mlp_user_hbm, mf_i_out, mlp_i_out, mf_u_out, mlp_u_out,
        idx_v, uidx_v, buf0, buf1, sem0, sem1):
    wid = lax.axis_index("s") * _NC + lax.axis_index("c")
    base = wid * (nch * ch)

    pltpu.sync_copy(item_idx_hbm.at[wid], idx_v)
    pltpu.sync_copy(user_idx_hbm.at[wid], uidx_v)

    bufs = (buf0, buf1)
    sems = (sem0, sem1)

    def do_table(table_hbm, out_hbm):
      cps = [None, None]
      cps[0] = pltpu.async_copy(table_hbm.at[idx_v.at[0]], bufs[0], sems[0])
      for j in range(1, nch + 1):
        if j < nch:
          cps[j % 2] = pltpu.async_copy(
              table_hbm.at[idx_v.at[j]], bufs[j % 2], sems[j % 2])
        p = (j - 1) % 2
        cps[p].wait()
        pltpu.sync_copy(bufs[p], out_hbm.at[pl.ds(base + (j - 1) * ch, ch)])

    do_table(mf_item_hbm, mf_i_out)
    do_table(mlp_item_hbm, mlp_i_out)

    ubase = wid * n_users_w
    cp = pltpu.async_copy(mf_user_hbm.at[uidx_v], buf0, sem0)
    cp.wait()
    pltpu.sync_copy(buf0.at[pl.ds(0, n_users_w)],
                    mf_u_out.at[pl.ds(ubase, n_users_w)])
    cp = pltpu.async_copy(mlp_user_hbm.at[uidx_v], buf1, sem1)
    cp.wait()
    pltpu.sync_copy(buf1.at[pl.ds(0, n_users_w)],
                    mlp_u_out.at[pl.ds(ubase, n_users_w)])

  return k(item_idx, user_idx, mf_item_em, mlp_item_em, mf_user_em,
           mlp_user_em)


def _tc_body(mf_i_ref, mlp_i_ref, mf_u_ref, mlp_u_ref, w1t_ref, w1b_ref,
             b1_ref, w2_ref, b2_ref, wmf_ref, wh2_ref, out_ref, *, bb, ll):
  f32 = jnp.float32
  a = jnp.dot(mlp_u_ref[...], w1t_ref[...], preferred_element_type=f32)
  a = a + b1_ref[...]
  vmf = mf_u_ref[...] * wmf_ref[...]
  r = lax.broadcasted_iota(jnp.int32, (bb * ll, bb), 0)
  c50 = lax.broadcasted_iota(jnp.int32, (bb * ll, bb), 1) * ll
  p = ((r >= c50) & (r < c50 + ll)).astype(f32)
  a_exp = jnp.dot(p, a, preferred_element_type=f32)
  vmf_exp = jnp.dot(p, vmf, preferred_element_type=f32)
  h1 = jnp.maximum(
      jnp.dot(mlp_i_ref[...], w1b_ref[...], preferred_element_type=f32)
      + a_exp, 0.0)
  h2 = jnp.maximum(
      jnp.dot(h1, w2_ref[...], preferred_element_type=f32) + b2_ref[...], 0.0)
  mf_c = jnp.sum(mf_i_ref[...] * vmf_exp, axis=1, keepdims=True)
  h2_c = jnp.sum(h2 * wh2_ref[...], axis=1, keepdims=True)
  out_ref[...] = mf_c + h2_c


def kernel(user, item, mf_user_em, mf_item_em, mlp_user_em, mlp_item_em,
           W1, b1, W2, b2, Wout):
  b, ll = item.shape
  d = mf_user_em.shape[1]
  bl = b * ll

  nch = bl // (_NW * _CH)
  item_idx = item.reshape(_NW, nch, _CH)
  user_idx = user.reshape(_NW, b // _NW)

  mf_i, mlp_i, mf_u, mlp_u = _sc_gather(
      item_idx, user_idx, mf_item_em, mlp_item_em, mf_user_em, mlp_user_em,
      nch)

  w1t = W1[:d, :]
  w1b = W1[d:, :]
  b1r = b1.reshape(1, d)
  b2r = b2.reshape(1, d // 2)
  wmf = Wout[:d, 0].reshape(1, d)
  wh2 = Wout[d:, 0].reshape(1, d // 2)

  bb = 64
  n_blocks = b // bb
  rows = bb * ll

  grid_spec = pl.GridSpec(
      grid=(n_blocks,),
      in_specs=[
          pl.BlockSpec((rows, d), lambda i: (i, 0)),
          pl.BlockSpec((rows, d), lambda i: (i, 0)),
          pl.BlockSpec((bb, d), lambda i: (i, 0)),
          pl.BlockSpec((bb, d), lambda i: (i, 0)),
          pl.BlockSpec((d, d), lambda i: (0, 0)),
          pl.BlockSpec((d, d), lambda i: (0, 0)),
          pl.BlockSpec((1, d), lambda i: (0, 0)),
          pl.BlockSpec((d, d // 2), lambda i: (0, 0)),
          pl.BlockSpec((1, d // 2), lambda i: (0, 0)),
          pl.BlockSpec((1, d), lambda i: (0, 0)),
          pl.BlockSpec((1, d // 2), lambda i: (0, 0)),
      ],
      out_specs=pl.BlockSpec((rows, 1), lambda i: (i, 0)),
  )

  out_flat = pl.pallas_call(
      functools.partial(_tc_body, bb=bb, ll=ll),
      grid_spec=grid_spec,
      out_shape=jax.ShapeDtypeStruct((bl, 1), jnp.float32),
      compiler_params=pltpu.CompilerParams(
          dimension_semantics=("parallel",)),
  )(mf_i, mlp_i, mf_u, mlp_u, w1t, w1b, b1r, W2, b2r, wmf, wh2)

  return out_flat.reshape(b, ll, 1)
